# baseline (speedup 1.0000x reference)
import functools

import jax
import jax.numpy as jnp
from jax import lax
from jax.experimental import pallas as pl
from jax.experimental.pallas import tpu as pltpu
from jax.experimental.pallas import tpu_sc as plsc

_BATCH = 16384
_DIM = 32
_V = 1000000
_NCHUNKS = 4
_CW = _BATCH // _NCHUNKS
_IPW = 8 * _CW

_mesh = plsc.VectorSubcoreMesh(core_axis_name="c", subcore_axis_name="s")


@functools.partial(
    pl.kernel,
    mesh=_mesh,
    out_type=jax.ShapeDtypeStruct((32 * _IPW,), jnp.float32),
    scratch_types=[
        pltpu.VMEM((_IPW,), jnp.int32),
        pltpu.VMEM((_IPW,), jnp.float32),
        pltpu.SemaphoreType.DMA,
    ],
)
def _rotate_lookup(tab1d, fidx, out, idx_v, val_v, sem):
    wid = lax.axis_index("s") * 2 + lax.axis_index("c")
    pltpu.sync_copy(fidx.at[pl.ds(wid * _IPW, _IPW)], idx_v)
    pltpu.async_copy(tab1d.at[idx_v], val_v, sem).wait()
    pltpu.sync_copy(val_v, out.at[pl.ds(wid * _IPW, _IPW)])


def kernel(entity_idx, ent_re, ent_im):
    idx = entity_idx.astype(jnp.int32)
    base = jnp.arange(2 * _DIM, dtype=jnp.int32) * _V
    fidx = (base.reshape(8, 1, 8, 1)
            + idx.reshape(1, _NCHUNKS, 1, _CW)).reshape(-1)
    tab1d = jnp.concatenate([ent_re.T.reshape(-1), ent_im.T.reshape(-1)])
    out = _rotate_lookup(tab1d, fidx)
    return (out.reshape(8, _NCHUNKS, 8, _CW)
            .transpose(0, 2, 1, 3).reshape(2 * _DIM, _BATCH).T)


# speedup vs baseline: 5.5944x; 5.5944x over previous
import functools

import jax
import jax.numpy as jnp
from jax import lax
from jax.experimental import pallas as pl
from jax.experimental.pallas import tpu as pltpu
from jax.experimental.pallas import tpu_sc as plsc

_BATCH = 16384
_DIM = 32
_V = 1000000
_NW = 32
_CW = _BATCH // _NW
_HPW = _DIM * _CW
_IPW = 2 * _HPW

_mesh = plsc.VectorSubcoreMesh(core_axis_name="c", subcore_axis_name="s")


@functools.partial(
    pl.kernel,
    mesh=_mesh,
    out_type=jax.ShapeDtypeStruct((_NW * _IPW,), jnp.float32),
    scratch_types=[
        pltpu.VMEM((_IPW,), jnp.int32),
        pltpu.VMEM((_HPW,), jnp.float32),
        pltpu.VMEM((_HPW,), jnp.float32),
        pltpu.SemaphoreType.DMA,
    ],
)
def _rotate_lookup(re1d, im1d, fidx, out, idx_v, val_re, val_im, sem):
    wid = lax.axis_index("s") * 2 + lax.axis_index("c")
    base = wid * _IPW
    pltpu.sync_copy(fidx.at[pl.ds(base, _IPW)], idx_v)
    c1 = pltpu.async_copy(re1d.at[idx_v.at[pl.ds(0, _HPW)]], val_re, sem)
    c2 = pltpu.async_copy(im1d.at[idx_v.at[pl.ds(_HPW, _HPW)]], val_im, sem)
    c1.wait()
    c2.wait()
    pltpu.sync_copy(val_re, out.at[pl.ds(base, _HPW)])
    pltpu.sync_copy(val_im, out.at[pl.ds(base + _HPW, _HPW)])


def kernel(entity_idx, ent_re, ent_im):
    idx = entity_idx.astype(jnp.int32)
    feat = jnp.arange(_DIM, dtype=jnp.int32)
    fidx = (idx.reshape(_NW, 1, 1, _CW) * _DIM
            + feat.reshape(1, 1, _DIM, 1))
    fidx = jnp.broadcast_to(fidx, (_NW, 2, _DIM, _CW)).reshape(-1)
    re1d = ent_re.reshape(-1)
    im1d = ent_im.reshape(-1)
    out = _rotate_lookup(re1d, im1d, fidx)
    return (out.reshape(_NW, 2, _DIM, _CW)
            .transpose(0, 3, 1, 2).reshape(_BATCH, 2 * _DIM))


# speedup vs baseline: 5.6041x; 1.0017x over previous
import functools

import jax
import jax.numpy as jnp
from jax import lax
from jax.experimental import pallas as pl
from jax.experimental.pallas import tpu as pltpu
from jax.experimental.pallas import tpu_sc as plsc

_BATCH = 16384
_DIM = 32
_V = 1000000
_NW = 32
_CW = _BATCH // _NW
_HPW = _DIM * _CW

_mesh = plsc.VectorSubcoreMesh(core_axis_name="c", subcore_axis_name="s")


@functools.partial(
    pl.kernel,
    mesh=_mesh,
    out_type=jax.ShapeDtypeStruct((2 * _DIM * _BATCH,), jnp.float32),
    scratch_types=[
        pltpu.VMEM((_CW,), jnp.int32),
        pltpu.VMEM((_HPW,), jnp.int32),
        pltpu.VMEM((_HPW,), jnp.float32),
        pltpu.VMEM((_HPW,), jnp.float32),
        pltpu.SemaphoreType.DMA,
    ],
)
def _rotate_lookup(re1d, im1d, idx, out, idx_c, off_v, val_re, val_im, sem):
    wid = lax.axis_index("s") * 2 + lax.axis_index("c")
    c0 = wid * _CW
    pltpu.sync_copy(idx.at[pl.ds(c0, _CW)], idx_c)
    for g in range(_CW // 16):
        iv = idx_c[pl.ds(g * 16, 16)] * _DIM
        for f in range(_DIM):
            off_v[pl.ds(f * _CW + g * 16, 16)] = iv + f
    c1 = pltpu.async_copy(re1d.at[off_v], val_re, sem)
    c2 = pltpu.async_copy(im1d.at[off_v], val_im, sem)
    c1.wait()
    c2.wait()
    for f in range(_DIM):
        pltpu.sync_copy(val_re.at[pl.ds(f * _CW, _CW)],
                        out.at[pl.ds(f * _BATCH + c0, _CW)])
        pltpu.sync_copy(val_im.at[pl.ds(f * _CW, _CW)],
                        out.at[pl.ds((_DIM + f) * _BATCH + c0, _CW)])


def kernel(entity_idx, ent_re, ent_im):
    idx = entity_idx.astype(jnp.int32)
    re1d = ent_re.reshape(-1)
    im1d = ent_im.reshape(-1)
    out = _rotate_lookup(re1d, im1d, idx)
    return out.reshape(2 * _DIM, _BATCH).T


# speedup vs baseline: 5.6419x; 1.0068x over previous
import functools

import jax
import jax.numpy as jnp
from jax import lax
from jax.experimental import pallas as pl
from jax.experimental.pallas import tpu as pltpu
from jax.experimental.pallas import tpu_sc as plsc

_BATCH = 16384
_DIM = 32
_V = 1000000
_NW = 32
_CW = _BATCH // _NW
_H = _CW // 2
_ROWS = _V * _DIM // 128

_mesh = plsc.VectorSubcoreMesh(core_axis_name="c", subcore_axis_name="s")


@functools.partial(
    pl.kernel,
    mesh=_mesh,
    out_type=jax.ShapeDtypeStruct((2 * _DIM * _BATCH,), jnp.float32),
    compiler_params=pltpu.CompilerParams(needs_layout_passes=False),
    scratch_types=[
        pltpu.VMEM((_CW,), jnp.int32),
        pltpu.VMEM((_H,), jnp.int32),
        pltpu.VMEM((_H, 128), jnp.float32),
        pltpu.VMEM((_H, 128), jnp.float32),
        pltpu.VMEM((_DIM * _H,), jnp.float32),
        pltpu.VMEM((_DIM * _H,), jnp.float32),
        pltpu.SemaphoreType.DMA,
    ],
)
def _rotate_lookup(re2d, im2d, idx, out, idx_c, row_v, rre, rim, vre, vim,
                   sem):
    wid = lax.axis_index("s") * 2 + lax.axis_index("c")
    c0 = wid * _CW
    pltpu.sync_copy(idx.at[pl.ds(c0, _CW)], idx_c)
    slot16 = lax.iota(jnp.int32, 16)
    for h in range(2):
        hb = h * _H
        for g in range(_H // 16):
            iv = idx_c[pl.ds(hb + g * 16, 16)]
            row_v[pl.ds(g * 16, 16)] = lax.shift_right_logical(iv, 2)
        c1 = pltpu.async_copy(re2d.at[row_v], rre, sem)
        c2 = pltpu.async_copy(im2d.at[row_v], rim, sem)
        c1.wait()
        c2.wait()

        def _extract(g, _):
            iv = idx_c[pl.ds(hb + g * 16, 16)]
            lane0 = (iv & 3) * _DIM
            rows = slot16 + g * 16
            for f in range(_DIM):
                lanes = lane0 + f
                vre[pl.ds(f * _H + g * 16, 16)] = plsc.load_gather(
                    rre, [rows, lanes])
                vim[pl.ds(f * _H + g * 16, 16)] = plsc.load_gather(
                    rim, [rows, lanes])
            return ()

        lax.fori_loop(0, _H // 16, _extract, ())
        for f in range(_DIM):
            pltpu.sync_copy(vre.at[pl.ds(f * _H, _H)],
                            out.at[pl.ds(f * _BATCH + c0 + hb, _H)])
            pltpu.sync_copy(vim.at[pl.ds(f * _H, _H)],
                            out.at[pl.ds((_DIM + f) * _BATCH + c0 + hb, _H)])


def kernel(entity_idx, ent_re, ent_im):
    idx = entity_idx.astype(jnp.int32)
    re2d = ent_re.reshape(_ROWS, 128)
    im2d = ent_im.reshape(_ROWS, 128)
    out = _rotate_lookup(re2d, im2d, idx)
    return out.reshape(2 * _DIM, _BATCH).T
